# baseline (speedup 1.0000x reference)
import functools

import jax
import jax.numpy as jnp
from jax import lax
from jax.experimental import pallas as pl
from jax.experimental.pallas import tpu as pltpu
from jax.experimental.pallas import tpu_sc as plsc

N = 50000
E = 800000
B = 8
DH = 64
HALF = 32
QW = 16
NC = 2
NS = 16

E_PAD1 = 917504
E_PAD2 = 851968
N_ACC = 53248
RPT = N_ACC // NS
CHUNK = 1024
IDXW = 128
ROWBLK = 2000
NSTEP = N // ROWBLK

def _edge_pipeline(gref, src_hbm, dst_hbm, accum, base_crow, ngroups,
                   src_v, dst_v, rows_v, SG, SS, SI):
    def fire_idx(crow, slot):
        pltpu.async_copy(src_hbm.at[pl.ds(crow, 8)], src_v.at[slot], SI[slot])
        pltpu.async_copy(dst_hbm.at[pl.ds(crow, 8)], dst_v.at[slot], SI[slot])

    def drain_i(slot):
        pltpu.make_async_copy(src_hbm.at[pl.ds(0, 8)], src_v.at[slot],
                              SI[slot]).wait()
        pltpu.make_async_copy(src_hbm.at[pl.ds(0, 8)], dst_v.at[slot],
                              SI[slot]).wait()

    def fire_gathers(slot):
        for j in range(8):
            pltpu.async_copy(gref.at[src_v.at[slot, j]],
                             rows_v.at[slot, pl.ds(j * IDXW, IDXW)], SG[slot])

    def drain_g(slot):
        pltpu.make_async_copy(gref.at[pl.ds(0, CHUNK)], rows_v.at[slot],
                              SG[slot]).wait()

    def fire_scatters(slot):
        for j in range(8):
            pltpu.async_copy(rows_v.at[slot, pl.ds(j * IDXW, IDXW)],
                             accum.at[dst_v.at[slot, j]], SS[slot], add=True)

    def drain_s(slot):
        pltpu.make_async_copy(gref.at[pl.ds(0, CHUNK)], rows_v.at[slot],
                              SS[slot]).wait()

    fire_idx(base_crow, 0)

    def group(g, carry):
        for k in range(4):
            kn = (k + 1) % 4
            kp = (k - 1) % 4
            drain_i(k)
            fire_gathers(k)
            if k == 3:
                drain_s(0)
            else:
                @pl.when(g >= 1)
                def _():
                    drain_s(kn)
            if k == 3:
                @pl.when(g < ngroups - 1)
                def _():
                    fire_idx(base_crow + (4 * (g + 1)) * 8, 0)
            else:
                fire_idx(base_crow + (4 * g + k + 1) * 8, kn)
            if k == 0:
                @pl.when(g >= 1)
                def _():
                    drain_g(kp)
                    fire_scatters(kp)
            else:
                drain_g(kp)
                fire_scatters(kp)
        return carry

    lax.fori_loop(0, ngroups, group, 0)
    drain_g(3)
    fire_scatters(3)
    drain_s(1)
    drain_s(2)
    drain_s(3)


def _agg1_body(x_hbm, src_hbm, dst_hbm, zeros_hbm, out_hbm,
               src_v, dst_v, rows_v, accum, *sems):
    c = lax.axis_index("c")
    s = lax.axis_index("s")
    w = c * NS + s
    pltpu.sync_copy(zeros_hbm, accum.at[pl.ds(s * RPT, RPT)])
    plsc.subcore_barrier()
    base_crow = w * (E_PAD1 // 32 // IDXW)
    _edge_pipeline(x_hbm, src_hbm, dst_hbm, accum, base_crow,
                   E_PAD1 // 32 // CHUNK // 4, src_v, dst_v, rows_v,
                   sems[0:4], sems[4:8], sems[8:12])
    plsc.subcore_barrier()
    pltpu.sync_copy(accum.at[pl.ds(s * RPT, RPT)],
                    out_hbm.at[c, pl.ds(s * RPT, RPT)])


def _agg2_body(hq_hbm, src_hbm, dst_hbm, zeros_hbm, out_hbm,
               src_v, dst_v, rows_v, accum, *sems):
    c = lax.axis_index("c")
    s = lax.axis_index("s")
    base_crow = s * (E_PAD2 // 16 // IDXW)

    for q in range(2):
        qi = c * 2 + q
        pltpu.sync_copy(zeros_hbm, accum.at[pl.ds(s * RPT, RPT)])
        plsc.subcore_barrier()
        _edge_pipeline(hq_hbm.at[qi], src_hbm, dst_hbm, accum, base_crow,
                       E_PAD2 // 16 // CHUNK // 4, src_v, dst_v, rows_v,
                       sems[0:4], sems[4:8], sems[8:12])
        plsc.subcore_barrier()
        pltpu.sync_copy(accum.at[pl.ds(s * RPT, RPT)],
                        out_hbm.at[qi, pl.ds(s * RPT, RPT)])


@functools.lru_cache(maxsize=None)
def _sc_aggs():
    mesh = plsc.VectorSubcoreMesh(core_axis_name="c", subcore_axis_name="s",
                                  num_cores=NC, num_subcores=NS)
    params = pltpu.CompilerParams(use_tc_tiling_on_sc=False)
    agg1 = pl.kernel(
        _agg1_body,
        out_type=jax.ShapeDtypeStruct((NC, N_ACC, QW), jnp.float32),
        mesh=mesh,
        compiler_params=params,
        scratch_types=[
            pltpu.VMEM((4, 8, IDXW), jnp.int32),
            pltpu.VMEM((4, 8, IDXW), jnp.int32),
            pltpu.VMEM((4, CHUNK, QW), jnp.float32),
            pltpu.VMEM_SHARED((N_ACC, QW), jnp.float32),
        ] + [pltpu.SemaphoreType.DMA] * 12,
    )
    agg2 = pl.kernel(
        _agg2_body,
        out_type=jax.ShapeDtypeStruct((4, N_ACC, QW), jnp.float32),
        mesh=mesh,
        compiler_params=params,
        scratch_types=[
            pltpu.VMEM((4, 8, IDXW), jnp.int32),
            pltpu.VMEM((4, 8, IDXW), jnp.int32),
            pltpu.VMEM((4, CHUNK, QW), jnp.float32),
            pltpu.VMEM_SHARED((N_ACC, QW), jnp.float32),
        ] + [pltpu.SemaphoreType.DMA] * 12,
    )
    return agg1, agg2


def _b1_body(x_ref, p0_ref, p1_ref, W1_ref, b1_ref, g1_ref, be1_ref,
             z_ref, s1_ref, t1_ref, ssum_ref, gram_ref, m0_ref):
    i = pl.program_id(0)
    z = x_ref[...] + p0_ref[0][:, :2] + p1_ref[0][:, :2]
    z_ref[...] = z

    @pl.when(i == 0)
    def _():
        m0_ref[...] = jnp.sum(z, axis=0, keepdims=True) / ROWBLK
        ssum_ref[...] = jnp.zeros_like(ssum_ref)
        gram_ref[...] = jnp.zeros_like(gram_ref)

    zc = z - m0_ref[...]
    ssum_ref[...] += jnp.sum(zc, axis=0, keepdims=True)
    gram_ref[...] += _fdot(zc, zc, (((0,), (0,)), ((), ())))

    @pl.when(i == NSTEP - 1)
    def _():
        meanc = ssum_ref[...] / N
        C = gram_ref[...] / N - _fdot(
            meanc, meanc, (((0,), (0,)), ((), ())))
        mean = meanc + m0_ref[...]
        W1 = W1_ref[...].astype(jnp.bfloat16).astype(jnp.float32)
        mu = _fdot(mean, W1, (((1,), (0,)), ((), ()))) + b1_ref[...]
        var = jnp.sum(_fdot(C, W1, (((1,), (0,)), ((), ()))) * W1,
                      axis=0, keepdims=True)
        s = g1_ref[...] * lax.rsqrt(var + 1e-5)
        s1_ref[...] = s
        t1_ref[...] = be1_ref[...] - mu * s


_HI = lax.Precision.HIGHEST


def _fdot(a, b, dims):
    return lax.dot_general(a, b, dims, precision=_HI)


def _bfdot(a, b):
    return jnp.dot(a.astype(jnp.bfloat16), b.astype(jnp.bfloat16),
                   preferred_element_type=jnp.float32)


def _b2_body(z_ref, batch_ref, W1_ref, b1_ref, s1_ref, t1_ref, W2_ref,
             b2_ref, hq_ref, h1p_ref, acc_ref):
    i = pl.program_id(0)
    z = z_ref[...]
    hp = _bfdot(z, W1_ref[...]) + b1_ref[...]
    h = jnp.maximum(hp * s1_ref[...] + t1_ref[...], 0.0)
    h1 = jnp.maximum(_bfdot(h, W2_ref[...]) + b2_ref[...], 0.0)
    hq_ref[0] = h1[:, 0 * QW:1 * QW]
    hq_ref[1] = h1[:, 1 * QW:2 * QW]
    hq_ref[2] = h1[:, 2 * QW:3 * QW]
    hq_ref[3] = h1[:, 3 * QW:4 * QW]
    bt = batch_ref[0]
    oh = (lax.broadcasted_iota(jnp.int32, (B, ROWBLK), 0) == bt
          ).astype(jnp.float32)
    pooled = _fdot(oh, h1, (((1,), (0,)), ((), ())))

    @pl.when(i == 0)
    def _():
        acc_ref[...] = jnp.zeros_like(acc_ref)

    acc_ref[...] += pooled

    @pl.when(i == NSTEP - 1)
    def _():
        h1p_ref[...] = acc_ref[...]


def _d1_body(q0_ref, q1_ref, q2_ref, q3_ref, a0_ref, a1_ref, a2_ref, a3_ref,
             W3_ref, b3_ref, g2_ref, be2_ref,
             u_ref, s2_ref, t2_ref, ssum_ref, gram_ref, m0_ref):
    i = pl.program_id(0)
    u = jnp.concatenate(
        [q0_ref[0] + a0_ref[0], q1_ref[0] + a1_ref[0],
         q2_ref[0] + a2_ref[0], q3_ref[0] + a3_ref[0]], axis=1)
    u_ref[...] = u

    @pl.when(i == 0)
    def _():
        m0_ref[...] = jnp.sum(u, axis=0, keepdims=True) / ROWBLK
        ssum_ref[...] = jnp.zeros_like(ssum_ref)
        gram_ref[...] = jnp.zeros_like(gram_ref)

    uc = u - m0_ref[...]
    ssum_ref[...] += jnp.sum(uc, axis=0, keepdims=True)
    gram_ref[...] += _fdot(uc, uc, (((0,), (0,)), ((), ())))

    @pl.when(i == NSTEP - 1)
    def _():
        meanc = ssum_ref[...] / N
        C = gram_ref[...] / N - _fdot(
            meanc, meanc, (((0,), (0,)), ((), ())))
        mean = meanc + m0_ref[...]
        W3 = W3_ref[...].astype(jnp.bfloat16).astype(jnp.float32)
        mu = _fdot(mean, W3, (((1,), (0,)), ((), ()))) + b3_ref[...]
        var = jnp.sum(_fdot(C, W3, (((1,), (0,)), ((), ()))) * W3,
                      axis=0, keepdims=True)
        s = g2_ref[...] * lax.rsqrt(var + 1e-5)
        s2_ref[...] = s
        t2_ref[...] = be2_ref[...] - mu * s


def _d2_body(u_ref, batch_ref, W3_ref, b3_ref, s2_ref, t2_ref, W4_ref,
             b4_ref, h2p_ref, acc_ref):
    i = pl.program_id(0)
    u = u_ref[...]
    hp = _bfdot(u, W3_ref[...]) + b3_ref[...]
    h = jnp.maximum(hp * s2_ref[...] + t2_ref[...], 0.0)
    h2 = jnp.maximum(_bfdot(h, W4_ref[...]) + b4_ref[...], 0.0)
    bt = batch_ref[0]
    oh = (lax.broadcasted_iota(jnp.int32, (B, ROWBLK), 0) == bt
          ).astype(jnp.float32)
    pooled = _fdot(oh, h2, (((1,), (0,)), ((), ())))

    @pl.when(i == 0)
    def _():
        acc_ref[...] = jnp.zeros_like(acc_ref)

    acc_ref[...] += pooled

    @pl.when(i == NSTEP - 1)
    def _():
        h2p_ref[...] = acc_ref[...]


def _head_body(h1p_ref, h2p_ref, st_ref, LA_ref, LB_ref, LC_ref, Lb1_ref,
               LW2_ref, Lb2_ref, out_ref):
    h = (_bfdot(h1p_ref[...], LA_ref[...]) + _bfdot(h2p_ref[...], LB_ref[...])
         + _bfdot(st_ref[...], LC_ref[...]) + Lb1_ref[...])
    h = jnp.maximum(h, 0.0)
    o = _bfdot(h, LW2_ref[...]) + Lb2_ref[...]
    m = jnp.max(o, axis=1, keepdims=True)
    lse = jnp.log(jnp.sum(jnp.exp(o - m), axis=1, keepdims=True)) + m
    out_ref[...] = o - lse


def _row_spec(cols):
    return pl.BlockSpec((ROWBLK, cols), lambda i: (i, 0))


def _full2(r, c):
    return pl.BlockSpec((r, c), lambda i: (0, 0))


def kernel(x, edge_index, batch, eig, stats, W1, b1, g1, be1, W2, b2,
           W3, b3, g2, be2, W4, b4, LW1, Lb1, LW2, Lb2):
    f32 = jnp.float32
    src = edge_index[0]
    dst = edge_index[1]
    pad = E_PAD1 - E
    srcp = jnp.concatenate(
        [src, jnp.zeros((pad,), jnp.int32)]).reshape(E_PAD1 // IDXW, IDXW)
    dstp = jnp.concatenate(
        [dst, N + jnp.arange(pad, dtype=jnp.int32) % (N_ACC - N)]
    ).reshape(E_PAD1 // IDXW, IDXW)
    zq = jnp.zeros((RPT, QW), f32)
    x16 = jnp.concatenate([x, jnp.zeros((N, QW - 2), f32)], axis=1)

    sc_agg1, sc_agg2 = _sc_aggs()
    agg1 = sc_agg1(x16, srcp, dstp, zq)

    z_arr, s1, t1 = pl.pallas_call(
        _b1_body,
        grid=(NSTEP,),
        in_specs=[
            _row_spec(2),
            pl.BlockSpec((1, ROWBLK, QW), lambda i: (0, i, 0)),
            pl.BlockSpec((1, ROWBLK, QW), lambda i: (1, i, 0)),
            _full2(2, DH), _full2(1, DH), _full2(1, DH), _full2(1, DH),
        ],
        out_specs=[_row_spec(2), _full2(1, DH), _full2(1, DH)],
        out_shape=[
            jax.ShapeDtypeStruct((N, 2), f32),
            jax.ShapeDtypeStruct((1, DH), f32),
            jax.ShapeDtypeStruct((1, DH), f32),
        ],
        scratch_shapes=[pltpu.VMEM((1, 2), f32), pltpu.VMEM((2, 2), f32),
                        pltpu.VMEM((1, 2), f32)],
    )(x, agg1, agg1, W1, b1.reshape(1, DH), g1.reshape(1, DH),
      be1.reshape(1, DH))

    batch3 = batch.reshape(NSTEP, 1, ROWBLK)
    hq, h1p = pl.pallas_call(
        _b2_body,
        grid=(NSTEP,),
        in_specs=[
            _row_spec(2),
            pl.BlockSpec((1, 1, ROWBLK), lambda i: (i, 0, 0)),
            _full2(2, DH), _full2(1, DH), _full2(1, DH), _full2(1, DH),
            _full2(DH, DH), _full2(1, DH),
        ],
        out_specs=[pl.BlockSpec((4, ROWBLK, QW), lambda i: (0, i, 0)),
                   _full2(B, DH)],
        out_shape=[jax.ShapeDtypeStruct((4, N, QW), f32),
                   jax.ShapeDtypeStruct((B, DH), f32)],
        scratch_shapes=[pltpu.VMEM((B, DH), f32)],
    )(z_arr, batch3, W1, b1.reshape(1, DH), s1, t1, W2, b2.reshape(1, DH))

    aq = sc_agg2(hq, srcp, dstp, zq)

    def _qspec(qi):
        return pl.BlockSpec((1, ROWBLK, QW), lambda i, qi=qi: (qi, i, 0))

    u, s2, t2 = pl.pallas_call(
        _d1_body,
        grid=(NSTEP,),
        in_specs=[_qspec(0), _qspec(1), _qspec(2), _qspec(3),
                  _qspec(0), _qspec(1), _qspec(2), _qspec(3),
                  _full2(DH, DH), _full2(1, DH), _full2(1, DH),
                  _full2(1, DH)],
        out_specs=[_row_spec(DH), _full2(1, DH), _full2(1, DH)],
        out_shape=[
            jax.ShapeDtypeStruct((N, DH), f32),
            jax.ShapeDtypeStruct((1, DH), f32),
            jax.ShapeDtypeStruct((1, DH), f32),
        ],
        scratch_shapes=[pltpu.VMEM((1, DH), f32), pltpu.VMEM((DH, DH), f32),
                        pltpu.VMEM((1, DH), f32)],
    )(hq, hq, hq, hq, aq, aq, aq, aq, W3,
      b3.reshape(1, DH), g2.reshape(1, DH), be2.reshape(1, DH))

    h2p = pl.pallas_call(
        _d2_body,
        grid=(NSTEP,),
        in_specs=[
            _row_spec(DH),
            pl.BlockSpec((1, 1, ROWBLK), lambda i: (i, 0, 0)),
            _full2(DH, DH), _full2(1, DH), _full2(1, DH), _full2(1, DH),
            _full2(DH, DH), _full2(1, DH),
        ],
        out_specs=_full2(B, DH),
        out_shape=jax.ShapeDtypeStruct((B, DH), f32),
        scratch_shapes=[pltpu.VMEM((B, DH), f32)],
    )(u, batch3, W3, b3.reshape(1, DH), s2, t2, W4, b4.reshape(1, DH))

    out = pl.pallas_call(
        _head_body,
        out_shape=jax.ShapeDtypeStruct((B, 5), f32),
    )(h1p, h2p, stats, LW1[:DH], LW1[DH:2 * DH], LW1[2 * DH:],
      Lb1.reshape(1, -1), LW2, Lb2.reshape(1, -1))
    return out


# speedup vs baseline: 2.7125x; 2.7125x over previous
import functools

import jax
import jax.numpy as jnp
from jax import lax
from jax.experimental import pallas as pl
from jax.experimental.pallas import tpu as pltpu
from jax.experimental.pallas import tpu_sc as plsc

N = 50000
E = 800000
B = 8
DH = 64
HALF = 32
QW = 16
NC = 2
NS = 16

E_PAD1 = 917504
E_PAD2 = 851968
N_ACC = 53248
RPT = N_ACC // NS
CHUNK = 1024
IDXW = 128
ROWBLK = 2000
NSTEP = N // ROWBLK

def _edge_pipeline(gref, src_hbm, dst_hbm, accum, base_crow, ngroups,
                   src_v, dst_v, rows_v, SG, SS, SI):
    def fire_idx(crow, slot):
        pltpu.async_copy(src_hbm.at[pl.ds(crow, 8)], src_v.at[slot], SI[slot])
        pltpu.async_copy(dst_hbm.at[pl.ds(crow, 8)], dst_v.at[slot], SI[slot])

    def drain_i(slot):
        pltpu.make_async_copy(src_hbm.at[pl.ds(0, 8)], src_v.at[slot],
                              SI[slot]).wait()
        pltpu.make_async_copy(src_hbm.at[pl.ds(0, 8)], dst_v.at[slot],
                              SI[slot]).wait()

    def fire_gathers(slot):
        for j in range(8):
            pltpu.async_copy(gref.at[src_v.at[slot, j]],
                             rows_v.at[slot, pl.ds(j * IDXW, IDXW)], SG[slot])

    def drain_g(slot):
        pltpu.make_async_copy(gref.at[pl.ds(0, CHUNK)], rows_v.at[slot],
                              SG[slot]).wait()

    def fire_scatters(slot):
        for j in range(8):
            pltpu.async_copy(rows_v.at[slot, pl.ds(j * IDXW, IDXW)],
                             accum.at[dst_v.at[slot, j]], SS[slot], add=True)

    def drain_s(slot):
        pltpu.make_async_copy(gref.at[pl.ds(0, CHUNK)], rows_v.at[slot],
                              SS[slot]).wait()

    fire_idx(base_crow, 0)

    def group(g, carry):
        for k in range(4):
            kn = (k + 1) % 4
            kp = (k - 1) % 4
            drain_i(k)
            fire_gathers(k)
            if k == 3:
                drain_s(0)
            else:
                @pl.when(g >= 1)
                def _():
                    drain_s(kn)
            if k == 3:
                @pl.when(g < ngroups - 1)
                def _():
                    fire_idx(base_crow + (4 * (g + 1)) * 8, 0)
            else:
                fire_idx(base_crow + (4 * g + k + 1) * 8, kn)
            if k == 0:
                @pl.when(g >= 1)
                def _():
                    drain_g(kp)
                    fire_scatters(kp)
            else:
                drain_g(kp)
                fire_scatters(kp)
        return carry

    lax.fori_loop(0, ngroups, group, 0)
    drain_g(3)
    fire_scatters(3)
    drain_s(1)
    drain_s(2)
    drain_s(3)


def _agg1_body(x_hbm, src_hbm, dst_hbm, zeros_hbm, out_hbm,
               src_v, dst_v, rows_v, accum, *sems):
    c = lax.axis_index("c")
    s = lax.axis_index("s")
    w = c * NS + s
    pltpu.sync_copy(zeros_hbm, accum.at[pl.ds(s * RPT, RPT)])
    plsc.subcore_barrier()
    base_crow = w * (E_PAD1 // 32 // IDXW)
    _edge_pipeline(x_hbm, src_hbm, dst_hbm, accum, base_crow,
                   E_PAD1 // 32 // CHUNK // 4, src_v, dst_v, rows_v,
                   sems[0:4], sems[4:8], sems[8:12])
    plsc.subcore_barrier()
    pltpu.sync_copy(accum.at[pl.ds(s * RPT, RPT)],
                    out_hbm.at[c, pl.ds(s * RPT, RPT)])


def _agg2_body(hq_hbm, src_hbm, dst_hbm, zeros_hbm, out_hbm,
               src_v, dst_v, rows_v, accum, *sems):
    c = lax.axis_index("c")
    s = lax.axis_index("s")
    base_crow = s * (E_PAD2 // 16 // IDXW)

    for q in range(2):
        qi = c * 2 + q
        pltpu.sync_copy(zeros_hbm, accum.at[pl.ds(s * RPT, RPT)])
        plsc.subcore_barrier()
        _edge_pipeline(hq_hbm.at[qi], src_hbm, dst_hbm, accum, base_crow,
                       E_PAD2 // 16 // CHUNK // 4, src_v, dst_v, rows_v,
                       sems[0:4], sems[4:8], sems[8:12])
        plsc.subcore_barrier()
        pltpu.sync_copy(accum.at[pl.ds(s * RPT, RPT)],
                        out_hbm.at[qi, pl.ds(s * RPT, RPT)])


@functools.lru_cache(maxsize=None)
def _sc_aggs():
    mesh = plsc.VectorSubcoreMesh(core_axis_name="c", subcore_axis_name="s",
                                  num_cores=NC, num_subcores=NS)
    params = pltpu.CompilerParams(use_tc_tiling_on_sc=False)
    agg1 = pl.kernel(
        _agg1_body,
        out_type=jax.ShapeDtypeStruct((NC, N_ACC, QW), jnp.float32),
        mesh=mesh,
        compiler_params=params,
        scratch_types=[
            pltpu.VMEM((4, 8, IDXW), jnp.int32),
            pltpu.VMEM((4, 8, IDXW), jnp.int32),
            pltpu.VMEM((4, CHUNK, QW), jnp.float32),
            pltpu.VMEM_SHARED((N_ACC, QW), jnp.float32),
        ] + [pltpu.SemaphoreType.DMA] * 12,
    )
    agg2 = pl.kernel(
        _agg2_body,
        out_type=jax.ShapeDtypeStruct((4, N_ACC, QW), jnp.float32),
        mesh=mesh,
        compiler_params=params,
        scratch_types=[
            pltpu.VMEM((4, 8, IDXW), jnp.int32),
            pltpu.VMEM((4, 8, IDXW), jnp.int32),
            pltpu.VMEM((4, CHUNK, QW), jnp.float32),
            pltpu.VMEM_SHARED((N_ACC, QW), jnp.float32),
        ] + [pltpu.SemaphoreType.DMA] * 12,
    )
    return agg1, agg2


def _b1_body(x_ref, p0_ref, p1_ref, W1_ref, b1_ref, g1_ref, be1_ref,
             z_ref, s1_ref, t1_ref, ssum_ref, gram_ref, m0_ref):
    i = pl.program_id(0)
    z = x_ref[...] + p0_ref[0][:, :2] + p1_ref[0][:, :2]
    z_ref[...] = z

    @pl.when(i == 0)
    def _():
        m0_ref[...] = jnp.sum(z, axis=0, keepdims=True) / ROWBLK
        ssum_ref[...] = jnp.zeros_like(ssum_ref)
        gram_ref[...] = jnp.zeros_like(gram_ref)

    zc = z - m0_ref[...]
    ssum_ref[...] += jnp.sum(zc, axis=0, keepdims=True)
    gram_ref[...] += _fdot(zc, zc, (((0,), (0,)), ((), ())))

    @pl.when(i == NSTEP - 1)
    def _():
        meanc = ssum_ref[...] / N
        C = gram_ref[...] / N - _fdot(
            meanc, meanc, (((0,), (0,)), ((), ())))
        mean = meanc + m0_ref[...]
        W1 = W1_ref[...].astype(jnp.bfloat16).astype(jnp.float32)
        mu = _fdot(mean, W1, (((1,), (0,)), ((), ()))) + b1_ref[...]
        var = jnp.sum(_fdot(C, W1, (((1,), (0,)), ((), ()))) * W1,
                      axis=0, keepdims=True)
        s = g1_ref[...] * lax.rsqrt(var + 1e-5)
        s1_ref[...] = s
        t1_ref[...] = be1_ref[...] - mu * s


_HI = lax.Precision.HIGHEST


def _fdot(a, b, dims):
    return lax.dot_general(a, b, dims, precision=_HI)


def _bfdot(a, b):
    return jnp.dot(a.astype(jnp.bfloat16), b.astype(jnp.bfloat16),
                   preferred_element_type=jnp.float32)


def _b2_body(z_ref, batch_ref, W1_ref, b1_ref, s1_ref, t1_ref, W2_ref,
             b2_ref, hq_ref, h1p_ref, acc_ref):
    i = pl.program_id(0)
    z = z_ref[...]
    hp = _bfdot(z, W1_ref[...]) + b1_ref[...]
    h = jnp.maximum(hp * s1_ref[...] + t1_ref[...], 0.0)
    h1 = jnp.maximum(_bfdot(h, W2_ref[...]) + b2_ref[...], 0.0)
    hq_ref[0] = h1[:, 0 * QW:1 * QW]
    hq_ref[1] = h1[:, 1 * QW:2 * QW]
    hq_ref[2] = h1[:, 2 * QW:3 * QW]
    hq_ref[3] = h1[:, 3 * QW:4 * QW]
    bt = batch_ref[0]
    oh = (lax.broadcasted_iota(jnp.int32, (B, ROWBLK), 0) == bt
          ).astype(jnp.float32)
    pooled = _fdot(oh, h1, (((1,), (0,)), ((), ())))

    @pl.when(i == 0)
    def _():
        acc_ref[...] = jnp.zeros_like(acc_ref)

    acc_ref[...] += pooled

    @pl.when(i == NSTEP - 1)
    def _():
        h1p_ref[...] = acc_ref[...]


def _d1_body(q0_ref, q1_ref, q2_ref, q3_ref, a0_ref, a1_ref, a2_ref, a3_ref,
             W3_ref, b3_ref, g2_ref, be2_ref,
             u_ref, s2_ref, t2_ref, ssum_ref, gram_ref, m0_ref):
    i = pl.program_id(0)
    u = jnp.concatenate(
        [q0_ref[0] + a0_ref[0], q1_ref[0] + a1_ref[0],
         q2_ref[0] + a2_ref[0], q3_ref[0] + a3_ref[0]], axis=1)
    u_ref[...] = u

    @pl.when(i == 0)
    def _():
        m0_ref[...] = jnp.sum(u, axis=0, keepdims=True) / ROWBLK
        ssum_ref[...] = jnp.zeros_like(ssum_ref)
        gram_ref[...] = jnp.zeros_like(gram_ref)

    uc = u - m0_ref[...]
    ssum_ref[...] += jnp.sum(uc, axis=0, keepdims=True)
    gram_ref[...] += _fdot(uc, uc, (((0,), (0,)), ((), ())))

    @pl.when(i == NSTEP - 1)
    def _():
        meanc = ssum_ref[...] / N
        C = gram_ref[...] / N - _fdot(
            meanc, meanc, (((0,), (0,)), ((), ())))
        mean = meanc + m0_ref[...]
        W3 = W3_ref[...].astype(jnp.bfloat16).astype(jnp.float32)
        mu = _fdot(mean, W3, (((1,), (0,)), ((), ()))) + b3_ref[...]
        var = jnp.sum(_fdot(C, W3, (((1,), (0,)), ((), ()))) * W3,
                      axis=0, keepdims=True)
        s = g2_ref[...] * lax.rsqrt(var + 1e-5)
        s2_ref[...] = s
        t2_ref[...] = be2_ref[...] - mu * s


def _d2_body(u_ref, batch_ref, W3_ref, b3_ref, s2_ref, t2_ref, W4_ref,
             b4_ref, h2p_ref, acc_ref):
    i = pl.program_id(0)
    u = u_ref[...]
    hp = _bfdot(u, W3_ref[...]) + b3_ref[...]
    h = jnp.maximum(hp * s2_ref[...] + t2_ref[...], 0.0)
    h2 = jnp.maximum(_bfdot(h, W4_ref[...]) + b4_ref[...], 0.0)
    bt = batch_ref[0]
    oh = (lax.broadcasted_iota(jnp.int32, (B, ROWBLK), 0) == bt
          ).astype(jnp.float32)
    pooled = _fdot(oh, h2, (((1,), (0,)), ((), ())))

    @pl.when(i == 0)
    def _():
        acc_ref[...] = jnp.zeros_like(acc_ref)

    acc_ref[...] += pooled

    @pl.when(i == NSTEP - 1)
    def _():
        h2p_ref[...] = acc_ref[...]


def _head_body(h1p_ref, h2p_ref, st_ref, LA_ref, LB_ref, LC_ref, Lb1_ref,
               LW2_ref, Lb2_ref, out_ref):
    h = (_bfdot(h1p_ref[...], LA_ref[...]) + _bfdot(h2p_ref[...], LB_ref[...])
         + _bfdot(st_ref[...], LC_ref[...]) + Lb1_ref[...])
    h = jnp.maximum(h, 0.0)
    o = _bfdot(h, LW2_ref[...]) + Lb2_ref[...]
    m = jnp.max(o, axis=1, keepdims=True)
    lse = jnp.log(jnp.sum(jnp.exp(o - m), axis=1, keepdims=True)) + m
    out_ref[...] = o - lse


def _row_spec(cols):
    return pl.BlockSpec((ROWBLK, cols), lambda i: (i, 0))


def _full2(r, c):
    return pl.BlockSpec((r, c), lambda i: (0, 0))


def kernel(x, edge_index, batch, eig, stats, W1, b1, g1, be1, W2, b2,
           W3, b3, g2, be2, W4, b4, LW1, Lb1, LW2, Lb2):
    f32 = jnp.float32
    src = edge_index[0]
    dst = edge_index[1]
    pad = E_PAD1 - E
    srcp = jnp.concatenate(
        [src, jnp.arange(pad, dtype=jnp.int32) * 61 % N]
    ).reshape(E_PAD1 // IDXW, IDXW)
    dstp = jnp.concatenate(
        [dst, N + jnp.arange(pad, dtype=jnp.int32) % (N_ACC - N)]
    ).reshape(E_PAD1 // IDXW, IDXW)
    zq = jnp.zeros((RPT, QW), f32)
    x16 = jnp.concatenate([x, jnp.zeros((N, QW - 2), f32)], axis=1)

    sc_agg1, sc_agg2 = _sc_aggs()
    agg1 = sc_agg1(x16, srcp, dstp, zq)

    z_arr, s1, t1 = pl.pallas_call(
        _b1_body,
        grid=(NSTEP,),
        in_specs=[
            _row_spec(2),
            pl.BlockSpec((1, ROWBLK, QW), lambda i: (0, i, 0)),
            pl.BlockSpec((1, ROWBLK, QW), lambda i: (1, i, 0)),
            _full2(2, DH), _full2(1, DH), _full2(1, DH), _full2(1, DH),
        ],
        out_specs=[_row_spec(2), _full2(1, DH), _full2(1, DH)],
        out_shape=[
            jax.ShapeDtypeStruct((N, 2), f32),
            jax.ShapeDtypeStruct((1, DH), f32),
            jax.ShapeDtypeStruct((1, DH), f32),
        ],
        scratch_shapes=[pltpu.VMEM((1, 2), f32), pltpu.VMEM((2, 2), f32),
                        pltpu.VMEM((1, 2), f32)],
    )(x, agg1, agg1, W1, b1.reshape(1, DH), g1.reshape(1, DH),
      be1.reshape(1, DH))

    batch3 = batch.reshape(NSTEP, 1, ROWBLK)
    hq, h1p = pl.pallas_call(
        _b2_body,
        grid=(NSTEP,),
        in_specs=[
            _row_spec(2),
            pl.BlockSpec((1, 1, ROWBLK), lambda i: (i, 0, 0)),
            _full2(2, DH), _full2(1, DH), _full2(1, DH), _full2(1, DH),
            _full2(DH, DH), _full2(1, DH),
        ],
        out_specs=[pl.BlockSpec((4, ROWBLK, QW), lambda i: (0, i, 0)),
                   _full2(B, DH)],
        out_shape=[jax.ShapeDtypeStruct((4, N, QW), f32),
                   jax.ShapeDtypeStruct((B, DH), f32)],
        scratch_shapes=[pltpu.VMEM((B, DH), f32)],
    )(z_arr, batch3, W1, b1.reshape(1, DH), s1, t1, W2, b2.reshape(1, DH))

    aq = sc_agg2(hq, srcp, dstp, zq)

    def _qspec(qi):
        return pl.BlockSpec((1, ROWBLK, QW), lambda i, qi=qi: (qi, i, 0))

    u, s2, t2 = pl.pallas_call(
        _d1_body,
        grid=(NSTEP,),
        in_specs=[_qspec(0), _qspec(1), _qspec(2), _qspec(3),
                  _qspec(0), _qspec(1), _qspec(2), _qspec(3),
                  _full2(DH, DH), _full2(1, DH), _full2(1, DH),
                  _full2(1, DH)],
        out_specs=[_row_spec(DH), _full2(1, DH), _full2(1, DH)],
        out_shape=[
            jax.ShapeDtypeStruct((N, DH), f32),
            jax.ShapeDtypeStruct((1, DH), f32),
            jax.ShapeDtypeStruct((1, DH), f32),
        ],
        scratch_shapes=[pltpu.VMEM((1, DH), f32), pltpu.VMEM((DH, DH), f32),
                        pltpu.VMEM((1, DH), f32)],
    )(hq, hq, hq, hq, aq, aq, aq, aq, W3,
      b3.reshape(1, DH), g2.reshape(1, DH), be2.reshape(1, DH))

    h2p = pl.pallas_call(
        _d2_body,
        grid=(NSTEP,),
        in_specs=[
            _row_spec(DH),
            pl.BlockSpec((1, 1, ROWBLK), lambda i: (i, 0, 0)),
            _full2(DH, DH), _full2(1, DH), _full2(1, DH), _full2(1, DH),
            _full2(DH, DH), _full2(1, DH),
        ],
        out_specs=_full2(B, DH),
        out_shape=jax.ShapeDtypeStruct((B, DH), f32),
        scratch_shapes=[pltpu.VMEM((B, DH), f32)],
    )(u, batch3, W3, b3.reshape(1, DH), s2, t2, W4, b4.reshape(1, DH))

    out = pl.pallas_call(
        _head_body,
        out_shape=jax.ShapeDtypeStruct((B, 5), f32),
    )(h1p, h2p, stats, LW1[:DH], LW1[DH:2 * DH], LW1[2 * DH:],
      Lb1.reshape(1, -1), LW2, Lb2.reshape(1, -1))
    return out
